# baseline (speedup 1.0000x reference)
import functools

import jax
import jax.numpy as jnp
from jax import lax
from jax.experimental import pallas as pl
from jax.experimental.pallas import tpu as pltpu
from jax.experimental.pallas import tpu_sc as plsc

IN_F = 128
OUT_F = 64
P = 1000
EPS = 1e-06
BATCH = 16384

_NC = 2
_NS = 16
_NW = _NC * _NS
_BPW = BATCH // _NW
_SB = 16
_NSB = _BPW // _SB
_TW = 2 * OUT_F


def _prep_body(x_ref, fr_ref, idx_ref):
    x = x_ref[...]
    mins = jnp.min(x, axis=0, keepdims=True)
    maxs = jnp.max(x, axis=0, keepdims=True)
    xs = (x - mins) / (maxs - mins + EPS) * (P - 1)
    low = jnp.clip(jnp.floor(xs), 0.0, P - 2)
    feat = lax.broadcasted_iota(jnp.int32, (BATCH, IN_F), 1)
    fr_ref[...] = xs - low
    idx_ref[...] = low.astype(jnp.int32) + feat * P


def _prep(x):
    return pl.pallas_call(
        _prep_body,
        out_shape=(
            jax.ShapeDtypeStruct((BATCH, IN_F), jnp.float32),
            jax.ShapeDtypeStruct((BATCH, IN_F), jnp.int32),
        ),
    )(x)


_FB = 8


def _table_body(r_ref, l_ref, tb_ref):
    row = lax.broadcasted_iota(jnp.int32, (P, P), 0)
    col = lax.broadcasted_iota(jnp.int32, (P, P), 1)
    tril = jnp.where(row >= col, 1.0, 0.0).astype(jnp.bfloat16)
    bias = lax.broadcasted_iota(jnp.int32, (P, OUT_F), 0).astype(jnp.float32)
    for i in range(_FB):
        r = r_ref[i]
        l = l_ref[i]
        u = r + l
        cat = jnp.concatenate([u, u * bias], axis=1)
        hi = cat.astype(jnp.bfloat16)
        lo = (cat - hi.astype(jnp.float32)).astype(jnp.bfloat16)
        cs = (jnp.dot(tril, hi, preferred_element_type=jnp.float32)
              + jnp.dot(tril, lo, preferred_element_type=jnp.float32))
        suml = jnp.sum(l, axis=0, keepdims=True)
        sumlb = jnp.sum(l * bias, axis=0, keepdims=True)
        s_part = cs[:, :OUT_F] - suml
        t_part = bias * s_part + (sumlb - cs[:, OUT_F:])
        t_u = lax.bitcast_convert_type(
            t_part.astype(jnp.bfloat16), jnp.uint16).astype(jnp.int32)
        s_u = lax.bitcast_convert_type(
            s_part.astype(jnp.bfloat16), jnp.uint16).astype(jnp.int32)
        tb_ref[i] = (t_u << 16) | s_u


def _table(r_weight, l_weight):
    return pl.pallas_call(
        _table_body,
        grid=(IN_F // _FB,),
        in_specs=[
            pl.BlockSpec((_FB, P, OUT_F), lambda f: (f, 0, 0)),
            pl.BlockSpec((_FB, P, OUT_F), lambda f: (f, 0, 0)),
        ],
        out_specs=pl.BlockSpec((_FB, P, OUT_F), lambda f: (f, 0, 0)),
        out_shape=jax.ShapeDtypeStruct((IN_F, P, OUT_F), jnp.int32),
    )(r_weight, l_weight)


def _sc_body(tb_hbm, idx_hbm, fr_hbm, out_hbm,
             idx_v, fr_v, buf0, buf1, buf2, buf3, out_v,
             gsem0, gsem1, gsem2, gsem3):
    wid = lax.axis_index("s") * _NC + lax.axis_index("c")
    row0 = wid * _BPW
    bufs = (buf0, buf1, buf2, buf3)
    gsems = (gsem0, gsem1, gsem2, gsem3)

    def issue(bl, k):
        pltpu.make_async_copy(tb_hbm.at[idx_v.at[bl]], bufs[k], gsems[k]).start()

    def drain(k):
        pltpu.make_async_copy(tb_hbm.at[pl.ds(0, IN_F)], bufs[k], gsems[k]).wait()

    def compute(bl, k):
        buf = bufs[k]

        def gbody(fg, accs):
            a0, a1, a2, a3 = accs
            fr16 = fr_v[pl.ds(bl * IN_F + fg * 16, 16)]
            for j in range(16):
                f = fg * 16 + j
                frv = jnp.broadcast_to(fr16[j], (16,))
                hm = jnp.int32(-65536)
                w0 = buf[f, pl.ds(0, 16)]
                w1 = buf[f, pl.ds(16, 16)]
                w2 = buf[f, pl.ds(32, 16)]
                w3 = buf[f, pl.ds(48, 16)]
                a0 = (a0 + lax.bitcast_convert_type(w0 & hm, jnp.float32)
                      + frv * lax.bitcast_convert_type(w0 << 16, jnp.float32))
                a1 = (a1 + lax.bitcast_convert_type(w1 & hm, jnp.float32)
                      + frv * lax.bitcast_convert_type(w1 << 16, jnp.float32))
                a2 = (a2 + lax.bitcast_convert_type(w2 & hm, jnp.float32)
                      + frv * lax.bitcast_convert_type(w2 << 16, jnp.float32))
                a3 = (a3 + lax.bitcast_convert_type(w3 & hm, jnp.float32)
                      + frv * lax.bitcast_convert_type(w3 << 16, jnp.float32))
            return (a0, a1, a2, a3)

        z = jnp.zeros((16,), jnp.float32)
        a0, a1, a2, a3 = lax.fori_loop(0, IN_F // 16, gbody, (z, z, z, z))
        out_v[bl, pl.ds(0, 16)] = a0
        out_v[bl, pl.ds(16, 16)] = a1
        out_v[bl, pl.ds(32, 16)] = a2
        out_v[bl, pl.ds(48, 16)] = a3

    def sb_body(s, carry):
        base = row0 + s * _SB
        pltpu.sync_copy(idx_hbm.at[pl.ds(base, _SB), :], idx_v)
        pltpu.sync_copy(fr_hbm.at[pl.ds(base * IN_F, _SB * IN_F)], fr_v)
        issue(0, 0)
        issue(1, 1)
        issue(2, 2)

        def quad_body(j, c):
            bl = 4 * j
            for t in range(4):
                drain(t)

                @pl.when(bl + t + 3 < _SB)
                def _():
                    issue(bl + t + 3, (t + 3) % 4)

                compute(bl + t, t)
            return c

        lax.fori_loop(0, _SB // 4, quad_body, 0)
        pltpu.sync_copy(out_v, out_hbm.at[pl.ds(base, _SB), :])
        return carry

    lax.fori_loop(0, _NSB, sb_body, 0)


_sc_kan = functools.partial(
    pl.kernel,
    out_type=jax.ShapeDtypeStruct((BATCH, OUT_F), jnp.float32),
    mesh=plsc.VectorSubcoreMesh(core_axis_name="c", subcore_axis_name="s",
                                num_cores=_NC, num_subcores=_NS),
    compiler_params=pltpu.CompilerParams(needs_layout_passes=False,
                                         use_tc_tiling_on_sc=False),
    scratch_types=[
        pltpu.VMEM((_SB, IN_F), jnp.int32),
        pltpu.VMEM((_SB * IN_F,), jnp.float32),
        pltpu.VMEM((IN_F, OUT_F), jnp.int32),
        pltpu.VMEM((IN_F, OUT_F), jnp.int32),
        pltpu.VMEM((IN_F, OUT_F), jnp.int32),
        pltpu.VMEM((IN_F, OUT_F), jnp.int32),
        pltpu.VMEM((_SB, OUT_F), jnp.float32),
        pltpu.SemaphoreType.DMA,
        pltpu.SemaphoreType.DMA,
        pltpu.SemaphoreType.DMA,
        pltpu.SemaphoreType.DMA,
    ],
)(_sc_body)


def kernel(x, r_weight, l_weight):
    fr, idx = _prep(x)
    tb = _table(r_weight, l_weight).reshape(IN_F * P, OUT_F)
    return _sc_kan(tb, idx, fr.reshape(-1))


# speedup vs baseline: 1.2799x; 1.2799x over previous
import functools

import jax
import jax.numpy as jnp
from jax import lax
from jax.experimental import pallas as pl
from jax.experimental.pallas import tpu as pltpu
from jax.experimental.pallas import tpu_sc as plsc

IN_F = 128
OUT_F = 64
P = 1000
EPS = 1e-06
BATCH = 16384

_NC = 2
_NS = 16
_NW = _NC * _NS
_BPW = BATCH // _NW
_SB = 16
_NSB = _BPW // _SB
_TW = 2 * OUT_F


def _prep_body(x_ref, fr_ref, idx_ref):
    x = x_ref[...]
    mins = jnp.min(x, axis=0, keepdims=True)
    maxs = jnp.max(x, axis=0, keepdims=True)
    xs = (x - mins) / (maxs - mins + EPS) * (P - 1)
    low = jnp.clip(jnp.floor(xs), 0.0, P - 2)
    feat = lax.broadcasted_iota(jnp.int32, (BATCH, IN_F), 1)
    fr_ref[...] = xs - low
    idx_ref[...] = low.astype(jnp.int32) + feat * P


def _prep(x):
    return pl.pallas_call(
        _prep_body,
        out_shape=(
            jax.ShapeDtypeStruct((BATCH, IN_F), jnp.float32),
            jax.ShapeDtypeStruct((BATCH, IN_F), jnp.int32),
        ),
    )(x)


_FB = 8


def _table_body(r_ref, l_ref, tb_ref):
    row = lax.broadcasted_iota(jnp.int32, (P, P), 0)
    col = lax.broadcasted_iota(jnp.int32, (P, P), 1)
    tril = jnp.where(row >= col, 1.0, 0.0).astype(jnp.bfloat16)
    bias = lax.broadcasted_iota(jnp.int32, (P, OUT_F), 0).astype(jnp.float32)
    for i in range(_FB):
        r = r_ref[i]
        l = l_ref[i]
        u = r + l
        cat = jnp.concatenate([u, u * bias], axis=1)
        hi = cat.astype(jnp.bfloat16)
        lo = (cat - hi.astype(jnp.float32)).astype(jnp.bfloat16)
        cs = (jnp.dot(tril, hi, preferred_element_type=jnp.float32)
              + jnp.dot(tril, lo, preferred_element_type=jnp.float32))
        suml = jnp.sum(l, axis=0, keepdims=True)
        sumlb = jnp.sum(l * bias, axis=0, keepdims=True)
        s_part = cs[:, :OUT_F] - suml
        t_part = bias * s_part + (sumlb - cs[:, OUT_F:])
        t_u = lax.bitcast_convert_type(
            (t_part * (1.0 - 2.0 ** -9)).astype(jnp.bfloat16),
            jnp.uint16).astype(jnp.int32)
        s_u = lax.bitcast_convert_type(
            s_part.astype(jnp.bfloat16), jnp.uint16).astype(jnp.int32)
        tb_ref[i] = (t_u << 16) | s_u


def _table(r_weight, l_weight):
    return pl.pallas_call(
        _table_body,
        grid=(IN_F // _FB,),
        in_specs=[
            pl.BlockSpec((_FB, P, OUT_F), lambda f: (f, 0, 0)),
            pl.BlockSpec((_FB, P, OUT_F), lambda f: (f, 0, 0)),
        ],
        out_specs=pl.BlockSpec((_FB, P, OUT_F), lambda f: (f, 0, 0)),
        out_shape=jax.ShapeDtypeStruct((IN_F, P, OUT_F), jnp.int32),
    )(r_weight, l_weight)


def _sc_body(tb_hbm, idx_hbm, fr_hbm, out_hbm,
             idx_v, fr_v, buf0, buf1, out_v, gsem0, gsem1):
    wid = lax.axis_index("s") * _NC + lax.axis_index("c")
    row0 = wid * _BPW
    bufs = (buf0, buf1)
    gsems = (gsem0, gsem1)

    def issue(bl, k):
        pltpu.make_async_copy(tb_hbm.at[idx_v.at[bl]], bufs[k], gsems[k]).start()

    def drain(k):
        pltpu.make_async_copy(tb_hbm.at[pl.ds(0, IN_F)], bufs[k], gsems[k]).wait()

    def compute(bl, k):
        buf = bufs[k]

        def gbody(fg, accs):
            a0, a1, a2, a3 = accs
            fr4 = fr_v[pl.ds(bl * IN_F + fg * 4, 16)]
            for j in range(4):
                f = fg * 4 + j
                frv = jnp.broadcast_to(fr4[j], (16,))
                w0 = buf[f, pl.ds(0, 16)]
                a0 = a0 + lax.bitcast_convert_type(w0, jnp.float32)
                a0 = a0 + frv * lax.bitcast_convert_type(w0 << 16, jnp.float32)
                w1 = buf[f, pl.ds(16, 16)]
                a1 = a1 + lax.bitcast_convert_type(w1, jnp.float32)
                a1 = a1 + frv * lax.bitcast_convert_type(w1 << 16, jnp.float32)
                w2 = buf[f, pl.ds(32, 16)]
                a2 = a2 + lax.bitcast_convert_type(w2, jnp.float32)
                a2 = a2 + frv * lax.bitcast_convert_type(w2 << 16, jnp.float32)
                w3 = buf[f, pl.ds(48, 16)]
                a3 = a3 + lax.bitcast_convert_type(w3, jnp.float32)
                a3 = a3 + frv * lax.bitcast_convert_type(w3 << 16, jnp.float32)
            return (a0, a1, a2, a3)

        z = jnp.zeros((16,), jnp.float32)
        a0, a1, a2, a3 = lax.fori_loop(0, IN_F // 4, gbody, (z, z, z, z))
        out_v[bl, pl.ds(0, 16)] = a0
        out_v[bl, pl.ds(16, 16)] = a1
        out_v[bl, pl.ds(32, 16)] = a2
        out_v[bl, pl.ds(48, 16)] = a3

    def sb_body(s, carry):
        base = row0 + s * _SB
        pltpu.sync_copy(idx_hbm.at[pl.ds(base, _SB), :], idx_v)
        pltpu.sync_copy(fr_hbm.at[pl.ds(base * IN_F, _SB * IN_F)],
                        fr_v.at[pl.ds(0, _SB * IN_F)])
        issue(0, 0)

        def pair_body(j, c):
            bl = 2 * j
            drain(0)
            issue(bl + 1, 1)
            compute(bl, 0)
            drain(1)

            @pl.when(bl + 2 < _SB)
            def _():
                issue(bl + 2, 0)

            compute(bl + 1, 1)
            return c

        lax.fori_loop(0, _SB // 2, pair_body, 0)
        pltpu.sync_copy(out_v, out_hbm.at[pl.ds(base, _SB), :])
        return carry

    lax.fori_loop(0, _NSB, sb_body, 0)


_sc_kan = functools.partial(
    pl.kernel,
    out_type=jax.ShapeDtypeStruct((BATCH, OUT_F), jnp.float32),
    mesh=plsc.VectorSubcoreMesh(core_axis_name="c", subcore_axis_name="s",
                                num_cores=_NC, num_subcores=_NS),
    compiler_params=pltpu.CompilerParams(needs_layout_passes=False,
                                         use_tc_tiling_on_sc=False),
    scratch_types=[
        pltpu.VMEM((_SB, IN_F), jnp.int32),
        pltpu.VMEM((_SB * IN_F + 16,), jnp.float32),
        pltpu.VMEM((IN_F, OUT_F), jnp.int32),
        pltpu.VMEM((IN_F, OUT_F), jnp.int32),
        pltpu.VMEM((_SB, OUT_F), jnp.float32),
        pltpu.SemaphoreType.DMA,
        pltpu.SemaphoreType.DMA,
    ],
)(_sc_body)


def kernel(x, r_weight, l_weight):
    fr, idx = _prep(x)
    tb = _table(r_weight, l_weight).reshape(IN_F * P, OUT_F)
    return _sc_kan(tb, idx, fr.reshape(-1))


# speedup vs baseline: 1.6581x; 1.2955x over previous
import functools

import jax
import jax.numpy as jnp
from jax import lax
from jax.experimental import pallas as pl
from jax.experimental.pallas import tpu as pltpu
from jax.experimental.pallas import tpu_sc as plsc

IN_F = 128
OUT_F = 64
P = 1000
EPS = 1e-06
BATCH = 16384

_NC = 2
_NS = 16
_NW = _NC * _NS
_BPW = BATCH // _NW
_SB = 16
_NSB = _BPW // _SB
_TW = 2 * OUT_F


def _prep_body(x_ref, fr_ref, idx_ref):
    x = x_ref[...]
    mins = jnp.min(x, axis=0, keepdims=True)
    maxs = jnp.max(x, axis=0, keepdims=True)
    xs = (x - mins) / (maxs - mins + EPS) * (P - 1)
    low = jnp.clip(jnp.floor(xs), 0.0, P - 2)
    feat = lax.broadcasted_iota(jnp.int32, (BATCH, IN_F), 1)
    fr_ref[...] = xs - low
    idx_ref[...] = low.astype(jnp.int32) + feat * P


def _prep(x):
    return pl.pallas_call(
        _prep_body,
        out_shape=(
            jax.ShapeDtypeStruct((BATCH, IN_F), jnp.float32),
            jax.ShapeDtypeStruct((BATCH, IN_F), jnp.int32),
        ),
    )(x)


_FB = 8


def _table_body(r_ref, l_ref, tb_ref):
    row = lax.broadcasted_iota(jnp.int32, (P, P), 0)
    col = lax.broadcasted_iota(jnp.int32, (P, P), 1)
    tril = jnp.where(row >= col, 1.0, 0.0).astype(jnp.bfloat16)
    bias = lax.broadcasted_iota(jnp.int32, (P, OUT_F), 0).astype(jnp.float32)
    for i in range(_FB):
        r = r_ref[i]
        l = l_ref[i]
        u = r + l
        cat = jnp.concatenate([u, u * bias], axis=1)
        hi = cat.astype(jnp.bfloat16)
        lo = (cat - hi.astype(jnp.float32)).astype(jnp.bfloat16)
        cs = (jnp.dot(tril, hi, preferred_element_type=jnp.float32)
              + jnp.dot(tril, lo, preferred_element_type=jnp.float32))
        suml = jnp.sum(l, axis=0, keepdims=True)
        sumlb = jnp.sum(l * bias, axis=0, keepdims=True)
        s_part = cs[:, :OUT_F] - suml
        t_part = bias * s_part + (sumlb - cs[:, OUT_F:])
        t_u = lax.bitcast_convert_type(
            (t_part * (1.0 - 2.0 ** -9)).astype(jnp.bfloat16),
            jnp.uint16).astype(jnp.int32)
        s_u = lax.bitcast_convert_type(
            s_part.astype(jnp.bfloat16), jnp.uint16).astype(jnp.int32)
        tb_ref[i] = (t_u << 16) | s_u


def _table(r_weight, l_weight):
    return pl.pallas_call(
        _table_body,
        grid=(IN_F // _FB,),
        in_specs=[
            pl.BlockSpec((_FB, P, OUT_F), lambda f: (f, 0, 0)),
            pl.BlockSpec((_FB, P, OUT_F), lambda f: (f, 0, 0)),
        ],
        out_specs=pl.BlockSpec((_FB, P, OUT_F), lambda f: (f, 0, 0)),
        out_shape=jax.ShapeDtypeStruct((IN_F, P, OUT_F), jnp.int32),
    )(r_weight, l_weight)


def _sc_body(tb_hbm, idx_hbm, fr_hbm, out_hbm,
             idx_v, fr_v, buf0, buf1, buf2, buf3, out_v,
             gsem0, gsem1, gsem2, gsem3):
    wid = lax.axis_index("s") * _NC + lax.axis_index("c")
    row0 = wid * _BPW
    bufs = (buf0, buf1, buf2, buf3)
    gsems = (gsem0, gsem1, gsem2, gsem3)

    def issue(bl, k):
        pltpu.make_async_copy(tb_hbm.at[idx_v.at[bl]], bufs[k], gsems[k]).start()

    def drain(k):
        pltpu.make_async_copy(tb_hbm.at[pl.ds(0, IN_F)], bufs[k], gsems[k]).wait()

    def compute(bl, k):
        buf = bufs[k]

        def gbody(fg, accs):
            a0, a1, a2, a3 = accs
            fr4 = fr_v[pl.ds(bl * IN_F + fg * 4, 16)]
            for j in range(4):
                f = fg * 4 + j
                frv = jnp.broadcast_to(fr4[j], (16,))
                w0 = buf[f, pl.ds(0, 16)]
                a0 = a0 + lax.bitcast_convert_type(w0, jnp.float32)
                a0 = a0 + frv * lax.bitcast_convert_type(w0 << 16, jnp.float32)
                w1 = buf[f, pl.ds(16, 16)]
                a1 = a1 + lax.bitcast_convert_type(w1, jnp.float32)
                a1 = a1 + frv * lax.bitcast_convert_type(w1 << 16, jnp.float32)
                w2 = buf[f, pl.ds(32, 16)]
                a2 = a2 + lax.bitcast_convert_type(w2, jnp.float32)
                a2 = a2 + frv * lax.bitcast_convert_type(w2 << 16, jnp.float32)
                w3 = buf[f, pl.ds(48, 16)]
                a3 = a3 + lax.bitcast_convert_type(w3, jnp.float32)
                a3 = a3 + frv * lax.bitcast_convert_type(w3 << 16, jnp.float32)
            return (a0, a1, a2, a3)

        z = jnp.zeros((16,), jnp.float32)
        a0, a1, a2, a3 = lax.fori_loop(0, IN_F // 4, gbody, (z, z, z, z))
        out_v[bl, pl.ds(0, 16)] = a0
        out_v[bl, pl.ds(16, 16)] = a1
        out_v[bl, pl.ds(32, 16)] = a2
        out_v[bl, pl.ds(48, 16)] = a3

    def sb_body(s, carry):
        base = row0 + s * _SB
        pltpu.sync_copy(idx_hbm.at[pl.ds(base, _SB), :], idx_v)
        pltpu.sync_copy(fr_hbm.at[pl.ds(base * IN_F, _SB * IN_F)],
                        fr_v.at[pl.ds(0, _SB * IN_F)])
        issue(0, 0)
        issue(1, 1)

        def quad_body(j, c):
            bl = 4 * j
            for t in range(4):
                drain(t)

                @pl.when(bl + t + 2 < _SB)
                def _():
                    issue(bl + t + 2, (t + 2) % 4)

                compute(bl + t, t)
            return c

        lax.fori_loop(0, _SB // 4, quad_body, 0)
        pltpu.sync_copy(out_v, out_hbm.at[pl.ds(base, _SB), :])
        return carry

    lax.fori_loop(0, _NSB, sb_body, 0)


_sc_kan = functools.partial(
    pl.kernel,
    out_type=jax.ShapeDtypeStruct((BATCH, OUT_F), jnp.float32),
    mesh=plsc.VectorSubcoreMesh(core_axis_name="c", subcore_axis_name="s",
                                num_cores=_NC, num_subcores=_NS),
    compiler_params=pltpu.CompilerParams(needs_layout_passes=False,
                                         use_tc_tiling_on_sc=False),
    scratch_types=[
        pltpu.VMEM((_SB, IN_F), jnp.int32),
        pltpu.VMEM((_SB * IN_F + 16,), jnp.float32),
        pltpu.VMEM((IN_F, OUT_F), jnp.int32),
        pltpu.VMEM((IN_F, OUT_F), jnp.int32),
        pltpu.VMEM((IN_F, OUT_F), jnp.int32),
        pltpu.VMEM((IN_F, OUT_F), jnp.int32),
        pltpu.VMEM((_SB, OUT_F), jnp.float32),
        pltpu.SemaphoreType.DMA,
        pltpu.SemaphoreType.DMA,
        pltpu.SemaphoreType.DMA,
        pltpu.SemaphoreType.DMA,
    ],
)(_sc_body)


def kernel(x, r_weight, l_weight):
    fr, idx = _prep(x)
    tb = _table(r_weight, l_weight).reshape(IN_F * P, OUT_F)
    return _sc_kan(tb, idx, fr.reshape(-1))


# speedup vs baseline: 1.6744x; 1.0098x over previous
import functools

import jax
import jax.numpy as jnp
from jax import lax
from jax.experimental import pallas as pl
from jax.experimental.pallas import tpu as pltpu
from jax.experimental.pallas import tpu_sc as plsc

IN_F = 128
OUT_F = 64
P = 1000
EPS = 1e-06
BATCH = 16384

_NC = 2
_NS = 16
_NW = _NC * _NS
_BPW = BATCH // _NW
_SB = 16
_NSB = _BPW // _SB
_TW = 2 * OUT_F


def _prep_body(x_ref, fr_ref, idx_ref):
    x = x_ref[...]
    mins = jnp.min(x, axis=0, keepdims=True)
    maxs = jnp.max(x, axis=0, keepdims=True)
    xs = (x - mins) / (maxs - mins + EPS) * (P - 1)
    low = jnp.clip(jnp.floor(xs), 0.0, P - 2)
    feat = lax.broadcasted_iota(jnp.int32, (BATCH, IN_F), 1)
    fr_ref[...] = xs - low
    idx_ref[...] = low.astype(jnp.int32) + feat * P


def _prep(x):
    return pl.pallas_call(
        _prep_body,
        out_shape=(
            jax.ShapeDtypeStruct((BATCH, IN_F), jnp.float32),
            jax.ShapeDtypeStruct((BATCH, IN_F), jnp.int32),
        ),
    )(x)


_FB = 8


def _table_body(r_ref, l_ref, tb_ref):
    row = lax.broadcasted_iota(jnp.int32, (P, P), 0)
    col = lax.broadcasted_iota(jnp.int32, (P, P), 1)
    tril = jnp.where(row >= col, 1.0, 0.0).astype(jnp.bfloat16)
    bias = lax.broadcasted_iota(jnp.int32, (P, OUT_F), 0).astype(jnp.float32)
    for i in range(_FB):
        r = r_ref[i]
        l = l_ref[i]
        u = r + l
        cat = jnp.concatenate([u, u * bias], axis=1)
        hi = cat.astype(jnp.bfloat16)
        lo = (cat - hi.astype(jnp.float32)).astype(jnp.bfloat16)
        cs = (jnp.dot(tril, hi, preferred_element_type=jnp.float32)
              + jnp.dot(tril, lo, preferred_element_type=jnp.float32))
        suml = jnp.sum(l, axis=0, keepdims=True)
        sumlb = jnp.sum(l * bias, axis=0, keepdims=True)
        s_part = cs[:, :OUT_F] - suml
        t_part = bias * s_part + (sumlb - cs[:, OUT_F:])
        t_u = lax.bitcast_convert_type(
            (t_part * (1.0 - 2.0 ** -9)).astype(jnp.bfloat16),
            jnp.uint16).astype(jnp.int32)
        s_u = lax.bitcast_convert_type(
            s_part.astype(jnp.bfloat16), jnp.uint16).astype(jnp.int32)
        tb_ref[i] = (t_u << 16) | s_u


def _table(r_weight, l_weight):
    return pl.pallas_call(
        _table_body,
        grid=(IN_F // _FB,),
        in_specs=[
            pl.BlockSpec((_FB, P, OUT_F), lambda f: (f, 0, 0)),
            pl.BlockSpec((_FB, P, OUT_F), lambda f: (f, 0, 0)),
        ],
        out_specs=pl.BlockSpec((_FB, P, OUT_F), lambda f: (f, 0, 0)),
        out_shape=jax.ShapeDtypeStruct((IN_F, P, OUT_F), jnp.int32),
    )(r_weight, l_weight)


def _sc_body(tb_hbm, idx_hbm, fr_hbm, out_hbm,
             idx_v, fr_v, buf0, buf1, buf2, buf3, out_v,
             gsem0, gsem1, gsem2, gsem3):
    wid = lax.axis_index("s") * _NC + lax.axis_index("c")
    row0 = wid * _BPW
    bufs = (buf0, buf1, buf2, buf3)
    gsems = (gsem0, gsem1, gsem2, gsem3)

    def issue(bl, k):
        pltpu.make_async_copy(tb_hbm.at[idx_v.at[bl]], bufs[k], gsems[k]).start()

    def drain(k):
        pltpu.make_async_copy(tb_hbm.at[pl.ds(0, IN_F)], bufs[k], gsems[k]).wait()

    def compute(bl, k):
        buf = bufs[k]

        def gbody(fg, accs):
            a0, a1, a2, a3 = accs
            fr4 = fr_v[pl.ds(bl * IN_F + fg * 4, 16)]
            for j in range(4):
                f = fg * 4 + j
                frv = jnp.broadcast_to(fr4[j], (16,))
                w0 = buf[f, pl.ds(0, 16)]
                a0 = a0 + lax.bitcast_convert_type(w0, jnp.float32)
                a0 = a0 + frv * lax.bitcast_convert_type(w0 << 16, jnp.float32)
                w1 = buf[f, pl.ds(16, 16)]
                a1 = a1 + lax.bitcast_convert_type(w1, jnp.float32)
                a1 = a1 + frv * lax.bitcast_convert_type(w1 << 16, jnp.float32)
                w2 = buf[f, pl.ds(32, 16)]
                a2 = a2 + lax.bitcast_convert_type(w2, jnp.float32)
                a2 = a2 + frv * lax.bitcast_convert_type(w2 << 16, jnp.float32)
                w3 = buf[f, pl.ds(48, 16)]
                a3 = a3 + lax.bitcast_convert_type(w3, jnp.float32)
                a3 = a3 + frv * lax.bitcast_convert_type(w3 << 16, jnp.float32)
            return (a0, a1, a2, a3)

        z = jnp.zeros((16,), jnp.float32)
        a0, a1, a2, a3 = lax.fori_loop(0, IN_F // 4, gbody, (z, z, z, z))
        out_v[bl, pl.ds(0, 16)] = a0
        out_v[bl, pl.ds(16, 16)] = a1
        out_v[bl, pl.ds(32, 16)] = a2
        out_v[bl, pl.ds(48, 16)] = a3

    def sb_body(s, carry):
        base = row0 + s * _SB
        pltpu.sync_copy(idx_hbm.at[pl.ds(base, _SB), :], idx_v)
        pltpu.sync_copy(fr_hbm.at[pl.ds(base * IN_F, _SB * IN_F)],
                        fr_v.at[pl.ds(0, _SB * IN_F)])
        issue(0, 0)
        issue(1, 1)
        issue(2, 2)

        def quad_body(j, c):
            bl = 4 * j
            for t in range(4):
                drain(t)

                @pl.when(bl + t + 3 < _SB)
                def _():
                    issue(bl + t + 3, (t + 3) % 4)

                compute(bl + t, t)
            return c

        lax.fori_loop(0, _SB // 4, quad_body, 0)
        pltpu.sync_copy(out_v, out_hbm.at[pl.ds(base, _SB), :])
        return carry

    lax.fori_loop(0, _NSB, sb_body, 0)


_sc_kan = functools.partial(
    pl.kernel,
    out_type=jax.ShapeDtypeStruct((BATCH, OUT_F), jnp.float32),
    mesh=plsc.VectorSubcoreMesh(core_axis_name="c", subcore_axis_name="s",
                                num_cores=_NC, num_subcores=_NS),
    compiler_params=pltpu.CompilerParams(needs_layout_passes=False,
                                         use_tc_tiling_on_sc=False),
    scratch_types=[
        pltpu.VMEM((_SB, IN_F), jnp.int32),
        pltpu.VMEM((_SB * IN_F + 16,), jnp.float32),
        pltpu.VMEM((IN_F, OUT_F), jnp.int32),
        pltpu.VMEM((IN_F, OUT_F), jnp.int32),
        pltpu.VMEM((IN_F, OUT_F), jnp.int32),
        pltpu.VMEM((IN_F, OUT_F), jnp.int32),
        pltpu.VMEM((_SB, OUT_F), jnp.float32),
        pltpu.SemaphoreType.DMA,
        pltpu.SemaphoreType.DMA,
        pltpu.SemaphoreType.DMA,
        pltpu.SemaphoreType.DMA,
    ],
)(_sc_body)


def kernel(x, r_weight, l_weight):
    fr, idx = _prep(x)
    tb = _table(r_weight, l_weight).reshape(IN_F * P, OUT_F)
    return _sc_kan(tb, idx, fr.reshape(-1))
